# baseline (speedup 1.0000x reference)
import math

import jax
import jax.numpy as jnp
from jax.experimental import pallas as pl
from jax.experimental.pallas import tpu as pltpu

_VOCAB = 32000
_PAD = 0
_SMOOTHING = 0.1
_CONF = 1.0 - _SMOOTHING
_EPS = _SMOOTHING / (_VOCAB - 2)
_K_CONST = _CONF * math.log(_CONF) + _SMOOTHING * math.log(_EPS)

_N = 4096
_BR = 32
_BC = 32000
_LW = 128


def _tc_body(t_ref, x_ref, out_ref, acc_ref):
    i = pl.program_id(0)
    j = pl.program_id(1)
    ni = pl.num_programs(0)
    nj = pl.num_programs(1)

    @pl.when(jnp.logical_and(i == 0, j == 0))
    def _init():
        acc_ref[...] = jnp.zeros_like(acc_ref)

    x = x_ref[...]
    t = t_ref[0, 0, :]
    validf = (t != _PAD).astype(jnp.float32)[:, None]

    lane = jax.lax.broadcasted_iota(jnp.int32, (_BR, _LW), 1)
    toff = t[:, None] - j * _BC

    rs_parts = []
    g_parts = []
    for w in range(_BC // _LW):
        xw = x[:, w * _LW:(w + 1) * _LW]
        rs_parts.append(xw)
        g_parts.append(jnp.where(toff - w * _LW == lane, xw, 0.0))

    def _tree(parts):
        while len(parts) > 1:
            nxt = [parts[k] + parts[k + 1] for k in range(0, len(parts) - 1, 2)]
            if len(parts) % 2:
                nxt.append(parts[-1])
            parts = nxt
        return parts[0]

    rs = _tree(rs_parts)
    g = _tree(g_parts)

    first = jnp.where(jnp.logical_and(lane == 0, j == 0),
                      _EPS * x[:, :_LW] + _K_CONST, 0.0)

    acc_ref[...] += (first - _EPS * rs - (_CONF - _EPS) * g) * validf

    @pl.when(jnp.logical_and(i == ni - 1, j == nj - 1))
    def _fin():
        out_ref[0, 0] = jnp.sum(acc_ref[...])


def kernel(pred, target):
    t3 = target.astype(jnp.int32).reshape(_N // _BR, 1, _BR)
    out = pl.pallas_call(
        _tc_body,
        grid=(_N // _BR, _VOCAB // _BC),
        in_specs=[
            pl.BlockSpec((1, 1, _BR), lambda i, j: (i, 0, 0)),
            pl.BlockSpec((_BR, _BC), lambda i, j: (i, j)),
        ],
        out_specs=pl.BlockSpec(memory_space=pltpu.SMEM),
        out_shape=jax.ShapeDtypeStruct((1, 1), jnp.float32),
        scratch_shapes=[pltpu.VMEM((_BR, _LW), jnp.float32)],
        compiler_params=pltpu.CompilerParams(
            dimension_semantics=("arbitrary", "arbitrary"),
        ),
    )(t3, pred)
    return out[0, 0]


# speedup vs baseline: 1.2763x; 1.2763x over previous
import math

import jax
import jax.numpy as jnp
from jax.experimental import pallas as pl
from jax.experimental.pallas import tpu as pltpu

_VOCAB = 32000
_PAD = 0
_SMOOTHING = 0.1
_CONF = 1.0 - _SMOOTHING
_EPS = _SMOOTHING / (_VOCAB - 2)
_K_CONST = _CONF * math.log(_CONF) + _SMOOTHING * math.log(_EPS)

_N = 4096
_BR = 128
_BC = 32000
_LW = 128


def _tc_body(t_ref, x_ref, out_ref, acc_ref):
    i = pl.program_id(0)
    j = pl.program_id(1)
    ni = pl.num_programs(0)
    nj = pl.num_programs(1)

    @pl.when(jnp.logical_and(i == 0, j == 0))
    def _init():
        acc_ref[...] = jnp.zeros_like(acc_ref)

    x = x_ref[...]
    t = t_ref[0, 0, :]
    validf = (t != _PAD).astype(jnp.float32)[:, None]

    lane = jax.lax.broadcasted_iota(jnp.int32, (_BR, _LW), 1)
    toff = t[:, None] - j * _BC

    rs_parts = []
    g_parts = []
    for w in range(_BC // _LW):
        xw = x[:, w * _LW:(w + 1) * _LW]
        rs_parts.append(xw)
        g_parts.append(jnp.where(toff - w * _LW == lane, xw, 0.0))

    def _tree(parts):
        while len(parts) > 1:
            nxt = [parts[k] + parts[k + 1] for k in range(0, len(parts) - 1, 2)]
            if len(parts) % 2:
                nxt.append(parts[-1])
            parts = nxt
        return parts[0]

    rs = _tree(rs_parts)
    g = _tree(g_parts)

    first = jnp.where(jnp.logical_and(lane == 0, j == 0),
                      _EPS * x[:, :_LW] + _K_CONST, 0.0)

    acc_ref[...] += (first - _EPS * rs - (_CONF - _EPS) * g) * validf

    @pl.when(jnp.logical_and(i == ni - 1, j == nj - 1))
    def _fin():
        out_ref[0, 0] = jnp.sum(acc_ref[...])


def kernel(pred, target):
    t3 = target.astype(jnp.int32).reshape(_N // _BR, 1, _BR)
    out = pl.pallas_call(
        _tc_body,
        grid=(_N // _BR, _VOCAB // _BC),
        in_specs=[
            pl.BlockSpec((1, 1, _BR), lambda i, j: (i, 0, 0)),
            pl.BlockSpec((_BR, _BC), lambda i, j: (i, j)),
        ],
        out_specs=pl.BlockSpec(memory_space=pltpu.SMEM),
        out_shape=jax.ShapeDtypeStruct((1, 1), jnp.float32),
        scratch_shapes=[pltpu.VMEM((_BR, _LW), jnp.float32)],
        compiler_params=pltpu.CompilerParams(
            dimension_semantics=("arbitrary", "arbitrary"),
        ),
    )(t3, pred)
    return out[0, 0]


# speedup vs baseline: 1.4505x; 1.1364x over previous
import math

import jax
import jax.numpy as jnp
from jax.experimental import pallas as pl
from jax.experimental.pallas import tpu as pltpu

_VOCAB = 32000
_PAD = 0
_SMOOTHING = 0.1
_CONF = 1.0 - _SMOOTHING
_EPS = _SMOOTHING / (_VOCAB - 2)
_K_CONST = _CONF * math.log(_CONF) + _SMOOTHING * math.log(_EPS)

_N = 4096
_BR = 128
_LW = 128
_NWIN = _VOCAB // _LW


def _tc_body(t_ref, x_ref, out_ref, acc_ref):
    i = pl.program_id(0)
    ni = pl.num_programs(0)

    @pl.when(i == 0)
    def _init():
        acc_ref[...] = jnp.zeros_like(acc_ref)

    t = t_ref[0, 0, :]
    validf = (t != _PAD).astype(jnp.float32)[:, None]
    lane = jax.lax.broadcasted_iota(jnp.int32, (_BR, _LW), 1)
    m = (t[:, None] & (_LW - 1)) == lane
    twin = t[:, None] >> 7

    rs = x_ref[:, 0:_LW]
    gw = rs
    for w in range(1, _NWIN):
        xw = x_ref[:, w * _LW:(w + 1) * _LW]
        rs = rs + xw
        gw = jnp.where(twin == w, xw, gw)
    g = jnp.where(m, gw, 0.0)

    first = jnp.where(lane == 0, _EPS * x_ref[:, 0:_LW] + _K_CONST, 0.0)
    acc_ref[...] += (first - _EPS * rs - (_CONF - _EPS) * g) * validf

    @pl.when(i == ni - 1)
    def _fin():
        out_ref[0, 0] = jnp.sum(acc_ref[...])


def kernel(pred, target):
    t3 = target.astype(jnp.int32).reshape(_N // _BR, 1, _BR)
    out = pl.pallas_call(
        _tc_body,
        grid=(_N // _BR,),
        in_specs=[
            pl.BlockSpec((1, 1, _BR), lambda i: (i, 0, 0)),
            pl.BlockSpec((_BR, _VOCAB), lambda i: (i, 0)),
        ],
        out_specs=pl.BlockSpec(memory_space=pltpu.SMEM),
        out_shape=jax.ShapeDtypeStruct((1, 1), jnp.float32),
        scratch_shapes=[pltpu.VMEM((_BR, _LW), jnp.float32)],
        compiler_params=pltpu.CompilerParams(
            dimension_semantics=("arbitrary",),
        ),
    )(t3, pred)
    return out[0, 0]
